# baseline (speedup 1.0000x reference)
import jax
import jax.numpy as jnp
from jax.experimental import pallas as pl

TOP_K = 512


def _copy_body(x_ref, o_ref):
    o_ref[...] = x_ref[...]


def kernel(inputs):
    pos = jnp.argsort(-inputs, axis=-1).astype(jnp.int32)
    pos = pl.pallas_call(
        _copy_body,
        out_shape=jax.ShapeDtypeStruct(pos.shape, pos.dtype),
    )(pos)
    return (pos[:, :TOP_K], pos[:, TOP_K:])


# speedup vs baseline: 1.3520x; 1.3520x over previous
import functools

import jax
import jax.numpy as jnp
from jax import lax
from jax.experimental import pallas as pl
from jax.experimental.pallas import tpu as pltpu
from jax.experimental.pallas import tpu_sc as plsc

ROWS = 64
N = 8192
TOP_K = 512
L = 16
CH = N // L
NB = 256
NW = 32
ROWS_PER_W = ROWS // NW


def _to_key(bits):
    m = lax.shift_right_arithmetic(bits, 31)
    return bits ^ (jnp.bitwise_not(m) & jnp.int32(0x7FFFFFFF))


def _digit(key, shift):
    return lax.shift_right_logical(key, shift) & jnp.int32(0xFF)


def _sort_body(in_hbm, sel_hbm, not_hbm, vals_v, key_a, idx_a, key_b, idx_b,
               hist_v, dma_sem):
    wid = lax.axis_index("s") * 2 + lax.axis_index("c")
    lanes = lax.iota(jnp.int32, L)
    g_base = lanes * CH
    ones = jnp.ones((L,), jnp.int32)
    zeros = jnp.zeros((L,), jnp.int32)

    def do_row(r, _):
        row = wid * ROWS_PER_W + r
        pltpu.sync_copy(in_hbm.at[row], vals_v)

        def init_step(t, _):
            sl = pl.ds(t * L, L)
            bits = lax.bitcast_convert_type(vals_v[sl] + jnp.float32(0.0),
                                            jnp.int32)
            key_a[sl] = _to_key(bits)
            return 0
        lax.fori_loop(0, N // L, init_step, 0)

        def do_pass(p, src_key, src_idx, dst_key, dst_idx, first):
            shift = p * 8

            def zero_step(i, _):
                hist_v[pl.ds(i * L, L)] = zeros
                return 0
            lax.fori_loop(0, NB * L // L, zero_step, 0)

            def hist_step(t, _):
                g = g_base + t
                k = plsc.load_gather(src_key, [g])
                d = _digit(k, shift)
                plsc.addupdate_scatter(hist_v, [d * L + lanes], ones)
                return 0
            lax.fori_loop(0, CH, hist_step, 0)

            def scan_step(i, carry):
                sl = pl.ds(i * L, L)
                h = hist_v[sl]
                inc = plsc.cumsum(h)
                hist_v[sl] = inc - h + carry
                return carry + jnp.sum(h)
            lax.fori_loop(0, NB, scan_step, jnp.int32(0))

            def scatter_step(t, _):
                g = g_base + t
                k = plsc.load_gather(src_key, [g])
                if first:
                    i = g
                else:
                    i = plsc.load_gather(src_idx, [g])
                d = _digit(k, shift)
                addr = d * L + lanes
                off = plsc.load_gather(hist_v, [addr])
                plsc.store_scatter(dst_key, [off], k)
                plsc.store_scatter(dst_idx, [off], i)
                plsc.addupdate_scatter(hist_v, [addr], ones)
                return 0
            lax.fori_loop(0, CH, scatter_step, 0)

        do_pass(0, key_a, None, key_b, idx_b, True)
        do_pass(1, key_b, idx_b, key_a, idx_a, False)
        do_pass(2, key_a, idx_a, key_b, idx_b, False)
        do_pass(3, key_b, idx_b, key_a, idx_a, False)

        pltpu.sync_copy(idx_a.at[pl.ds(0, TOP_K)], sel_hbm.at[row])
        pltpu.sync_copy(idx_a.at[pl.ds(TOP_K, N - TOP_K)], not_hbm.at[row])
        return 0

    lax.fori_loop(0, ROWS_PER_W, do_row, 0)


@jax.jit
def _run(inputs):
    mesh = plsc.VectorSubcoreMesh(core_axis_name="c", subcore_axis_name="s")
    f = pl.kernel(
        _sort_body,
        out_type=(
            jax.ShapeDtypeStruct((ROWS, TOP_K), jnp.int32),
            jax.ShapeDtypeStruct((ROWS, N - TOP_K), jnp.int32),
        ),
        mesh=mesh,
        scratch_types=[
            pltpu.VMEM((N,), jnp.float32),
            pltpu.VMEM((N,), jnp.int32),
            pltpu.VMEM((N,), jnp.int32),
            pltpu.VMEM((N,), jnp.int32),
            pltpu.VMEM((N,), jnp.int32),
            pltpu.VMEM((NB * L,), jnp.int32),
            pltpu.SemaphoreType.DMA,
        ],
        compiler_params=pltpu.CompilerParams(needs_layout_passes=False),
    )
    return f(inputs)


def kernel(inputs):
    return _run(inputs)
